# baseline (speedup 1.0000x reference)
import functools

import jax
import jax.numpy as jnp
from jax import lax
from jax.experimental import pallas as pl
from jax.experimental.pallas import tpu as pltpu
from jax.experimental.pallas import tpu_sc as plsc

NUM_T1 = 6
NUM_T2 = 3
NUM_COMBO = NUM_T1 * NUM_T2
D = 128
E = 320000

NC = 2
NS = 16
NW = NC * NS
EPW = E // NW
CH = 80
NCHUNK = EPW // CH
G16 = CH // 16


def _combo_body(t1_ref, t2_ref, out_ref):
    r = lax.broadcasted_iota(jnp.int32, (NUM_COMBO, D), 0)
    a = r // NUM_T2
    b = r - a * NUM_T2
    acc = jnp.zeros((NUM_COMBO, D), jnp.float32)
    for i in range(NUM_T1):
        acc = jnp.where(a == i, t1_ref[i, :][None, :], acc)
    for j in range(NUM_T2):
        acc = acc + jnp.where(b == j, t2_ref[j, :][None, :], 0.0)
    out_ref[...] = acc


_combo_tc = pl.pallas_call(
    _combo_body,
    out_shape=jax.ShapeDtypeStruct((NUM_COMBO, D), jnp.float32),
)


def _sc_body(combo_hbm, attr_hbm, out_hbm, attr_v, idx_v, buf_v, gsem):
    wid = lax.axis_index("s") * NC + lax.axis_index("c")
    base = wid * EPW

    pltpu.sync_copy(attr_hbm.at[pl.ds(2 * base, 2 * EPW)], attr_v)

    lanes = lax.iota(jnp.int32, 16)

    def compute_idx(r, carry):
        for j in range(G16):
            le = (r * CH + j * 16 + lanes) * 2
            a = plsc.load_gather(attr_v, [le])
            b = plsc.load_gather(attr_v, [le + 1])
            idx_v[r, pl.ds(j * 16, 16)] = a * NUM_T2 + b
        return carry

    lax.fori_loop(0, NCHUNK, compute_idx, 0)

    def gather_store(t, carry):
        pltpu.async_copy(combo_hbm.at[idx_v.at[t]], buf_v, gsem).wait()
        pltpu.sync_copy(buf_v, out_hbm.at[pl.ds(base + t * CH, CH), :])
        return carry

    lax.fori_loop(0, NCHUNK, gather_store, 0)


_sc_gather = functools.partial(
    pl.kernel,
    out_type=jax.ShapeDtypeStruct((E, D), jnp.float32),
    mesh=plsc.VectorSubcoreMesh(core_axis_name="c", subcore_axis_name="s"),
    compiler_params=pltpu.CompilerParams(needs_layout_passes=False),
    scratch_types=[
        pltpu.VMEM((2 * EPW,), jnp.int32),
        pltpu.VMEM((NCHUNK, CH), jnp.int32),
        pltpu.VMEM((CH, D), jnp.float32),
        pltpu.SemaphoreType.DMA,
    ],
)(_sc_body)


@jax.jit
def kernel(edge_attr, table1, table2):
    combo = _combo_tc(table1, table2)
    return _sc_gather(combo, edge_attr.reshape(-1))


# speedup vs baseline: 1.0268x; 1.0268x over previous
import functools

import jax
import jax.numpy as jnp
from jax import lax
from jax.experimental import pallas as pl
from jax.experimental.pallas import tpu as pltpu
from jax.experimental.pallas import tpu_sc as plsc

NUM_T1 = 6
NUM_T2 = 3
NUM_COMBO = NUM_T1 * NUM_T2
D = 128
E = 320000

NC = 2
NS = 16
NW = NC * NS
EPW = E // NW
BUFE = 400
NG16 = BUFE // 16
NGROUP = EPW // BUFE


def _combo_body(t1_ref, t2_ref, out_ref):
    r = lax.broadcasted_iota(jnp.int32, (NUM_COMBO, D), 0)
    a = r // NUM_T2
    b = r - a * NUM_T2
    acc = jnp.zeros((NUM_COMBO, D), jnp.float32)
    for i in range(NUM_T1):
        acc = jnp.where(a == i, t1_ref[i, :][None, :], acc)
    for j in range(NUM_T2):
        acc = acc + jnp.where(b == j, t2_ref[j, :][None, :], 0.0)
    out_ref[...] = acc


_combo_tc = pl.pallas_call(
    _combo_body,
    out_shape=jax.ShapeDtypeStruct((NUM_COMBO, D), jnp.float32),
)


def _sc_body(combo_hbm, attr_hbm, out_hbm, attr_v, buf_v,
             gsem, wsem0, wsem1):
    wid = lax.axis_index("s") * NC + lax.axis_index("c")
    base = wid * EPW

    pltpu.sync_copy(attr_hbm.at[pl.ds(2 * base, 2 * EPW)], attr_v)

    lanes = lax.iota(jnp.int32, 16)

    def group(g, carry):
        p = lax.rem(g, 2)

        @pl.when(jnp.logical_and(g >= 2, p == 0))
        def _():
            pltpu.make_async_copy(
                buf_v.at[0], out_hbm.at[pl.ds(base, BUFE), :], wsem0).wait()

        @pl.when(jnp.logical_and(g >= 2, p == 1))
        def _():
            pltpu.make_async_copy(
                buf_v.at[1], out_hbm.at[pl.ds(base, BUFE), :], wsem1).wait()

        def fire(k, carry2):
            le = (g * BUFE + k * 16 + lanes) * 2
            a = plsc.load_gather(attr_v, [le])
            b = plsc.load_gather(attr_v, [le + 1])
            idx16 = a * NUM_T2 + b
            pltpu.async_copy(
                combo_hbm.at[idx16],
                buf_v.at[p, pl.ds(k * 16, 16), :], gsem)
            return carry2

        lax.fori_loop(0, NG16, fire, 0)

        def drain(k, carry2):
            pltpu.make_async_copy(
                combo_hbm.at[lanes],
                buf_v.at[p, pl.ds(0, 16), :], gsem).wait()
            return carry2

        lax.fori_loop(0, NG16, drain, 0)

        @pl.when(p == 0)
        def _():
            pltpu.async_copy(
                buf_v.at[0],
                out_hbm.at[pl.ds(base + g * BUFE, BUFE), :], wsem0)

        @pl.when(p == 1)
        def _():
            pltpu.async_copy(
                buf_v.at[1],
                out_hbm.at[pl.ds(base + g * BUFE, BUFE), :], wsem1)

        return carry

    lax.fori_loop(0, NGROUP, group, 0)

    pltpu.make_async_copy(
        buf_v.at[0], out_hbm.at[pl.ds(base, BUFE), :],
        wsem1 if (NGROUP - 2) % 2 else wsem0).wait()
    pltpu.make_async_copy(
        buf_v.at[0], out_hbm.at[pl.ds(base, BUFE), :],
        wsem0 if (NGROUP - 2) % 2 else wsem1).wait()


_sc_gather = functools.partial(
    pl.kernel,
    out_type=jax.ShapeDtypeStruct((E, D), jnp.float32),
    mesh=plsc.VectorSubcoreMesh(core_axis_name="c", subcore_axis_name="s"),
    compiler_params=pltpu.CompilerParams(needs_layout_passes=False),
    scratch_types=[
        pltpu.VMEM((2 * EPW,), jnp.int32),
        pltpu.VMEM((2, BUFE, D), jnp.float32),
        pltpu.SemaphoreType.DMA,
        pltpu.SemaphoreType.DMA,
        pltpu.SemaphoreType.DMA,
    ],
)(_sc_body)


@jax.jit
def kernel(edge_attr, table1, table2):
    combo = _combo_tc(table1, table2)
    return _sc_gather(combo, edge_attr.reshape(-1))


# speedup vs baseline: 6.9261x; 6.7452x over previous
import functools

import jax
import jax.numpy as jnp
from jax import lax
from jax.experimental import pallas as pl
from jax.experimental.pallas import tpu as pltpu
from jax.experimental.pallas import tpu_sc as plsc

NUM_T1 = 6
NUM_T2 = 3
NUM_COMBO = NUM_T1 * NUM_T2
D = 128
E = 320000

NC = 2
NS = 16
NW = NC * NS
EPW = E // NW
BUFE = 400
NG16 = BUFE // 16
NGROUP = EPW // BUFE


def _combo_body(t1_ref, t2_ref, out_ref):
    r = lax.broadcasted_iota(jnp.int32, (NUM_COMBO, D), 0)
    a = r // NUM_T2
    b = r - a * NUM_T2
    acc = jnp.zeros((NUM_COMBO, D), jnp.float32)
    for i in range(NUM_T1):
        acc = jnp.where(a == i, t1_ref[i, :][None, :], acc)
    for j in range(NUM_T2):
        acc = acc + jnp.where(b == j, t2_ref[j, :][None, :], 0.0)
    out_ref[...] = acc


_combo_tc = pl.pallas_call(
    _combo_body,
    out_shape=jax.ShapeDtypeStruct((NUM_COMBO, D), jnp.float32),
)


def _sc_body(combo_hbm, attr_hbm, out_hbm, attr_v, buf_v, combo_sh,
             gsem, wsem0, wsem1):
    sid = lax.axis_index("s")
    wid = sid * NC + lax.axis_index("c")
    base = wid * EPW

    @pl.when(sid == 0)
    def _():
        pltpu.sync_copy(combo_hbm, combo_sh)

    pltpu.sync_copy(attr_hbm.at[pl.ds(2 * base, 2 * EPW)], attr_v)
    plsc.subcore_barrier()

    lanes = lax.iota(jnp.int32, 16)

    def group(g, carry):
        p = lax.rem(g, 2)

        @pl.when(jnp.logical_and(g >= 2, p == 0))
        def _():
            pltpu.make_async_copy(
                buf_v.at[0], out_hbm.at[pl.ds(base, BUFE), :], wsem0).wait()

        @pl.when(jnp.logical_and(g >= 2, p == 1))
        def _():
            pltpu.make_async_copy(
                buf_v.at[1], out_hbm.at[pl.ds(base, BUFE), :], wsem1).wait()

        def fire(k, carry2):
            le = (g * BUFE + k * 16 + lanes) * 2
            a = plsc.load_gather(attr_v, [le])
            b = plsc.load_gather(attr_v, [le + 1])
            idx16 = a * NUM_T2 + b
            pltpu.async_copy(
                combo_sh.at[idx16],
                buf_v.at[p, pl.ds(k * 16, 16), :], gsem)
            return carry2

        lax.fori_loop(0, NG16, fire, 0)

        def drain(k, carry2):
            pltpu.make_async_copy(
                combo_sh.at[lanes],
                buf_v.at[p, pl.ds(0, 16), :], gsem).wait()
            return carry2

        lax.fori_loop(0, NG16, drain, 0)

        @pl.when(p == 0)
        def _():
            pltpu.async_copy(
                buf_v.at[0],
                out_hbm.at[pl.ds(base + g * BUFE, BUFE), :], wsem0)

        @pl.when(p == 1)
        def _():
            pltpu.async_copy(
                buf_v.at[1],
                out_hbm.at[pl.ds(base + g * BUFE, BUFE), :], wsem1)

        return carry

    lax.fori_loop(0, NGROUP, group, 0)

    pltpu.make_async_copy(
        buf_v.at[0], out_hbm.at[pl.ds(base, BUFE), :],
        wsem1 if (NGROUP - 2) % 2 else wsem0).wait()
    pltpu.make_async_copy(
        buf_v.at[0], out_hbm.at[pl.ds(base, BUFE), :],
        wsem0 if (NGROUP - 2) % 2 else wsem1).wait()


_sc_gather = functools.partial(
    pl.kernel,
    out_type=jax.ShapeDtypeStruct((E, D), jnp.float32),
    mesh=plsc.VectorSubcoreMesh(core_axis_name="c", subcore_axis_name="s"),
    compiler_params=pltpu.CompilerParams(needs_layout_passes=False),
    scratch_types=[
        pltpu.VMEM((2 * EPW,), jnp.int32),
        pltpu.VMEM((2, BUFE, D), jnp.float32),
        pltpu.VMEM_SHARED((NUM_COMBO, D), jnp.float32),
        pltpu.SemaphoreType.DMA,
        pltpu.SemaphoreType.DMA,
        pltpu.SemaphoreType.DMA,
    ],
)(_sc_body)


@jax.jit
def kernel(edge_attr, table1, table2):
    combo = _combo_tc(table1, table2)
    return _sc_gather(combo, edge_attr.reshape(-1))


# speedup vs baseline: 20.1982x; 2.9162x over previous
import functools

import jax
import jax.numpy as jnp
from jax import lax
from jax.experimental import pallas as pl
from jax.experimental.pallas import tpu as pltpu
from jax.experimental.pallas import tpu_sc as plsc

NUM_T1 = 6
NUM_T2 = 3
NUM_COMBO = NUM_T1 * NUM_T2
D = 128
E = 320000

NC = 2
NS = 16
NW = NC * NS
EPW = E // NW
BUFE = 400
NG16 = BUFE // 16
NGROUP = EPW // BUFE


def _combo_body(t1_ref, t2_ref, out_ref):
    r = lax.broadcasted_iota(jnp.int32, (NUM_COMBO, D), 0)
    a = r // NUM_T2
    b = r - a * NUM_T2
    acc = jnp.zeros((NUM_COMBO, D), jnp.float32)
    for i in range(NUM_T1):
        acc = jnp.where(a == i, t1_ref[i, :][None, :], acc)
    for j in range(NUM_T2):
        acc = acc + jnp.where(b == j, t2_ref[j, :][None, :], 0.0)
    out_ref[...] = acc


_combo_tc = pl.pallas_call(
    _combo_body,
    out_shape=jax.ShapeDtypeStruct((NUM_COMBO, D), jnp.float32),
)


def _sc_body(combo_hbm, attr_hbm, out_hbm, attr_v, buf_v, combo_sh,
             gsem, wsem0, wsem1):
    sid = lax.axis_index("s")
    wid = sid * NC + lax.axis_index("c")
    base = wid * EPW

    @pl.when(sid == 0)
    def _():
        pltpu.sync_copy(combo_hbm, combo_sh)

    pltpu.sync_copy(attr_hbm.at[pl.ds(base, EPW)], attr_v.at[pl.ds(0, EPW)])
    pltpu.sync_copy(attr_hbm.at[pl.ds(E + base, EPW)],
                    attr_v.at[pl.ds(EPW, EPW)])
    plsc.subcore_barrier()

    lanes = lax.iota(jnp.int32, 16)

    def group(g, carry):
        p = lax.rem(g, 2)

        @pl.when(jnp.logical_and(g >= 2, p == 0))
        def _():
            pltpu.make_async_copy(
                buf_v.at[0], out_hbm.at[pl.ds(base, BUFE), :], wsem0).wait()

        @pl.when(jnp.logical_and(g >= 2, p == 1))
        def _():
            pltpu.make_async_copy(
                buf_v.at[1], out_hbm.at[pl.ds(base, BUFE), :], wsem1).wait()

        def fire(k, carry2):
            le = g * BUFE + k * 16 + lanes
            a = plsc.load_gather(attr_v, [le])
            b = plsc.load_gather(attr_v, [le + EPW])
            idx16 = a * NUM_T2 + b
            pltpu.async_copy(
                combo_sh.at[idx16],
                buf_v.at[p, pl.ds(k * 16, 16), :], gsem)
            return carry2

        lax.fori_loop(0, NG16, fire, 0)

        def drain(k, carry2):
            pltpu.make_async_copy(
                combo_sh.at[lanes],
                buf_v.at[p, pl.ds(0, 16), :], gsem).wait()
            return carry2

        lax.fori_loop(0, NG16, drain, 0)

        @pl.when(p == 0)
        def _():
            pltpu.async_copy(
                buf_v.at[0],
                out_hbm.at[pl.ds(base + g * BUFE, BUFE), :], wsem0)

        @pl.when(p == 1)
        def _():
            pltpu.async_copy(
                buf_v.at[1],
                out_hbm.at[pl.ds(base + g * BUFE, BUFE), :], wsem1)

        return carry

    lax.fori_loop(0, NGROUP, group, 0)

    pltpu.make_async_copy(
        buf_v.at[0], out_hbm.at[pl.ds(base, BUFE), :],
        wsem1 if (NGROUP - 2) % 2 else wsem0).wait()
    pltpu.make_async_copy(
        buf_v.at[0], out_hbm.at[pl.ds(base, BUFE), :],
        wsem0 if (NGROUP - 2) % 2 else wsem1).wait()


_sc_gather = functools.partial(
    pl.kernel,
    out_type=jax.ShapeDtypeStruct((E, D), jnp.float32),
    mesh=plsc.VectorSubcoreMesh(core_axis_name="c", subcore_axis_name="s"),
    compiler_params=pltpu.CompilerParams(needs_layout_passes=False),
    scratch_types=[
        pltpu.VMEM((2 * EPW,), jnp.int32),
        pltpu.VMEM((2, BUFE, D), jnp.float32),
        pltpu.VMEM_SHARED((NUM_COMBO, D), jnp.float32),
        pltpu.SemaphoreType.DMA,
        pltpu.SemaphoreType.DMA,
        pltpu.SemaphoreType.DMA,
    ],
)(_sc_body)


@jax.jit
def kernel(edge_attr, table1, table2):
    combo = _combo_tc(table1, table2)
    return _sc_gather(combo, jnp.ravel(edge_attr.T))
